# Initial kernel scaffold; baseline (speedup 1.0000x reference)
import jax
import jax.numpy as jnp
from jax.experimental import pallas as pl


def kernel(x, batch_lengths):
    raise NotImplementedError("write your pallas kernel here")


# speedup vs baseline: 2.9589x; 2.9589x over previous
import jax
import jax.numpy as jnp
from jax import lax
from jax.experimental import pallas as pl
from jax.experimental.pallas import tpu as pltpu
from jax.experimental.pallas import tpu_sc as plsc

_N = 32768
_B = 16
_D = 256
_NC = 2
_NS = 16
_NW = _NC * _NS
_L = 16
_C = 256
_DV = _D // _L


def _lane_select(vec, s):
    lane = lax.iota(jnp.int32, _L)
    return jnp.max(jnp.where(lane == s, vec, 0))


def _sum_body(x_hbm, len_hbm, part_hbm, len_v, buf, acc):
    cid = lax.axis_index("c")
    sid = lax.axis_index("s")
    wid = sid * _NC + cid

    pltpu.sync_copy(len_hbm, len_v)
    lens = len_v[...]
    csum = plsc.cumsum(lens)
    total = jnp.max(csum)

    zero = jnp.zeros((_L,), jnp.float32)

    def zbody(i, c):
        acc[pl.ds(i * _L, _L)] = zero
        return c

    lax.fori_loop(0, _B * _D // _L, zbody, 0)

    offs = [jnp.int32(0)] + [_lane_select(csum, s) for s in range(_B)]

    nchunks = (total + _C - 1) // _C
    kw = (nchunks - wid + _NW - 1) // _NW

    def chunk_body(k, c):
        g = wid + k * _NW
        row0 = g * _C
        pltpu.sync_copy(x_hbm.at[pl.ds(row0 * _D, _C * _D)], buf)
        row1 = jnp.minimum(row0 + _C, total)
        for s in range(_B):
            a = jnp.maximum(row0, offs[s])
            b = jnp.minimum(row1, offs[s + 1])

            @pl.when(b > a)
            def _():
                def rbody(r, carry):
                    base = (r - row0) * _D
                    return tuple(
                        carry[j] + buf[pl.ds(base + j * _L, _L)]
                        for j in range(_DV)
                    )

                run = lax.fori_loop(a, b, rbody, (zero,) * _DV)
                for j in range(_DV):
                    o = s * _D + j * _L
                    acc[pl.ds(o, _L)] = acc[pl.ds(o, _L)] + run[j]

        return c

    lax.fori_loop(0, kw, chunk_body, 0)
    pltpu.sync_copy(acc, part_hbm.at[pl.ds(wid * _B * _D, _B * _D)])


def _combine_body(part_hbm, len_hbm, out_hbm, idx_v, rows_v, len_v, out_v, sem):
    cid = lax.axis_index("c")
    sid = lax.axis_index("s")
    wid = sid * _NC + cid

    @pl.when(wid < _B)
    def _():
        lane = lax.iota(jnp.int32, _L)
        idx_v[pl.ds(0, _L)] = lane * _B + wid
        idx_v[pl.ds(_L, _L)] = (lane + _L) * _B + wid
        pltpu.async_copy(part_hbm.at[idx_v], rows_v, sem).wait()

        pltpu.sync_copy(len_hbm, len_v)
        lens = len_v[...]
        cnt = jnp.max(jnp.where(lane == wid, jnp.maximum(lens, 1), 0))
        cnt_vec = jnp.full((_L,), cnt, jnp.int32).astype(jnp.float32)
        recip = jnp.ones((_L,), jnp.float32) / cnt_vec

        def rbody(r, carry):
            return tuple(
                carry[j] + rows_v[r, pl.ds(j * _L, _L)] for j in range(_DV)
            )

        tot = lax.fori_loop(
            0, _NW, rbody, (jnp.zeros((_L,), jnp.float32),) * _DV
        )
        for j in range(_DV):
            out_v[pl.ds(j * _L, _L)] = tot[j] * recip
        pltpu.sync_copy(out_v, out_hbm.at[pl.ds(wid * _D, _D)])


_mesh = plsc.VectorSubcoreMesh(core_axis_name="c", subcore_axis_name="s")
_params = pltpu.CompilerParams(needs_layout_passes=False)

_sum_call = pl.kernel(
    _sum_body,
    out_type=jax.ShapeDtypeStruct((_NW * _B * _D,), jnp.float32),
    mesh=_mesh,
    compiler_params=_params,
    scratch_types=[
        pltpu.VMEM((_L,), jnp.int32),
        pltpu.VMEM((_C * _D,), jnp.float32),
        pltpu.VMEM((_B * _D,), jnp.float32),
    ],
)

_combine_call = pl.kernel(
    _combine_body,
    out_type=jax.ShapeDtypeStruct((_B * _D,), jnp.float32),
    mesh=_mesh,
    compiler_params=_params,
    scratch_types=[
        pltpu.VMEM((_NW,), jnp.int32),
        pltpu.VMEM((_NW, _D), jnp.float32),
        pltpu.VMEM((_L,), jnp.int32),
        pltpu.VMEM((_D,), jnp.float32),
        pltpu.SemaphoreType.DMA,
    ],
)


def kernel(x, batch_lengths):
    part = _sum_call(x.reshape(-1), batch_lengths)
    out = _combine_call(part.reshape(_NW * _B, _D), batch_lengths)
    return out.reshape(_B, _D)


# speedup vs baseline: 4.8413x; 1.6362x over previous
import jax
import jax.numpy as jnp
from jax import lax
from jax.experimental import pallas as pl
from jax.experimental.pallas import tpu as pltpu
from jax.experimental.pallas import tpu_sc as plsc

_N = 32768
_B = 16
_D = 256
_NC = 2
_NS = 16
_NW = _NC * _NS
_L = 16
_C = 256
_DV = _D // _L


def _lane_select(vec, s):
    lane = lax.iota(jnp.int32, _L)
    return jnp.max(jnp.where(lane == s, vec, 0))


def _sum_body(x_hbm, len_hbm, part_hbm, len_v, buf, acc):
    cid = lax.axis_index("c")
    sid = lax.axis_index("s")
    wid = sid * _NC + cid

    pltpu.sync_copy(len_hbm, len_v)
    lens = len_v[...]
    csum = plsc.cumsum(lens)
    total = jnp.max(csum)

    zero = jnp.zeros((_L,), jnp.float32)

    for s in range(_B):
        for j in range(_DV):
            acc[s, pl.ds(j * _L, _L)] = zero

    offs = [jnp.int32(0)] + [_lane_select(csum, s) for s in range(_B)]

    nchunks = (total + _C - 1) // _C
    kw = (nchunks - wid + _NW - 1) // _NW

    def chunk_body(k, c):
        g = wid + k * _NW
        row0 = g * _C
        pltpu.sync_copy(x_hbm.at[pl.ds(row0, _C), :], buf)
        row1 = jnp.minimum(row0 + _C, total)
        for s in range(_B):
            a = jnp.maximum(row0, offs[s])
            b = jnp.minimum(row1, offs[s + 1])

            @pl.when(b > a)
            def _():
                def rbody(r, carry):
                    rr = r - row0
                    return tuple(
                        carry[j] + buf[rr, pl.ds(j * _L, _L)]
                        for j in range(_DV)
                    )

                run = lax.fori_loop(a, b, rbody, (zero,) * _DV)
                for j in range(_DV):
                    o = j * _L
                    acc[s, pl.ds(o, _L)] = acc[s, pl.ds(o, _L)] + run[j]

        return c

    lax.fori_loop(0, kw, chunk_body, 0)
    pltpu.sync_copy(acc, part_hbm.at[pl.ds(wid * _B, _B), :])


def _combine_body(part_hbm, len_hbm, out_hbm, idx_v, rows_v, len_v, out_v, sem):
    cid = lax.axis_index("c")
    sid = lax.axis_index("s")
    wid = sid * _NC + cid

    @pl.when(wid < _B)
    def _():
        lane = lax.iota(jnp.int32, _L)
        idx_v[pl.ds(0, _L)] = lane * _B + wid
        idx_v[pl.ds(_L, _L)] = (lane + _L) * _B + wid
        pltpu.async_copy(part_hbm.at[idx_v], rows_v, sem).wait()

        pltpu.sync_copy(len_hbm, len_v)
        lens = len_v[...]
        cnt = jnp.max(jnp.where(lane == wid, jnp.maximum(lens, 1), 0))
        cnt_vec = jnp.full((_L,), cnt, jnp.int32).astype(jnp.float32)
        recip = jnp.ones((_L,), jnp.float32) / cnt_vec

        def rbody(r, carry):
            return tuple(
                carry[j] + rows_v[r, pl.ds(j * _L, _L)] for j in range(_DV)
            )

        tot = lax.fori_loop(
            0, _NW, rbody, (jnp.zeros((_L,), jnp.float32),) * _DV
        )
        for j in range(_DV):
            out_v[0, pl.ds(j * _L, _L)] = tot[j] * recip
        pltpu.sync_copy(out_v, out_hbm.at[pl.ds(wid, 1), :])


_mesh = plsc.VectorSubcoreMesh(core_axis_name="c", subcore_axis_name="s")
_params = pltpu.CompilerParams(needs_layout_passes=False)

_sum_call = pl.kernel(
    _sum_body,
    out_type=jax.ShapeDtypeStruct((_NW * _B, _D), jnp.float32),
    mesh=_mesh,
    compiler_params=_params,
    scratch_types=[
        pltpu.VMEM((_L,), jnp.int32),
        pltpu.VMEM((_C, _D), jnp.float32),
        pltpu.VMEM((_B, _D), jnp.float32),
    ],
)

_combine_call = pl.kernel(
    _combine_body,
    out_type=jax.ShapeDtypeStruct((_B, _D), jnp.float32),
    mesh=_mesh,
    compiler_params=_params,
    scratch_types=[
        pltpu.VMEM((_NW,), jnp.int32),
        pltpu.VMEM((_NW, _D), jnp.float32),
        pltpu.VMEM((_L,), jnp.int32),
        pltpu.VMEM((1, _D), jnp.float32),
        pltpu.SemaphoreType.DMA,
    ],
)


def kernel(x, batch_lengths):
    part = _sum_call(x, batch_lengths)
    return _combine_call(part, batch_lengths)


# speedup vs baseline: 4.9925x; 1.0312x over previous
import jax
import jax.numpy as jnp
from jax import lax
from jax.experimental import pallas as pl
from jax.experimental.pallas import tpu as pltpu
from jax.experimental.pallas import tpu_sc as plsc

_N = 32768
_B = 16
_D = 256
_NC = 2
_NS = 16
_NW = _NC * _NS
_L = 16
_C = 128
_DV = _D // _L


def _lane_select(vec, s):
    lane = lax.iota(jnp.int32, _L)
    return jnp.max(jnp.where(lane == s, vec, 0))


def _sum_body(x_hbm, len_hbm, part_hbm, len_v, buf0, buf1, acc, sem0, sem1):
    cid = lax.axis_index("c")
    sid = lax.axis_index("s")
    wid = sid * _NC + cid

    pltpu.sync_copy(len_hbm, len_v)
    lens = len_v[...]
    csum = plsc.cumsum(lens)
    total = jnp.max(csum)

    zero = jnp.zeros((_L,), jnp.float32)

    for s in range(_B):
        for j in range(_DV):
            acc[s, pl.ds(j * _L, _L)] = zero

    offs = [jnp.int32(0)] + [_lane_select(csum, s) for s in range(_B)]

    nchunks = (total + _C - 1) // _C
    kw = (nchunks - wid + _NW - 1) // _NW

    bufs = (buf0, buf1)
    sems = (sem0, sem1)

    def copy_of(k, slot):
        row0 = (wid + k * _NW) * _C
        return pltpu.make_async_copy(
            x_hbm.at[pl.ds(row0, _C), :], bufs[slot], sems[slot]
        )

    @pl.when(kw > 0)
    def _():
        copy_of(0, 0).start()

    def process(k, slot):
        buf = bufs[slot]
        row0 = (wid + k * _NW) * _C

        @pl.when(k + 1 < kw)
        def _():
            copy_of(k + 1, 1 - slot).start()

        copy_of(k, slot).wait()
        row1 = jnp.minimum(row0 + _C, total)
        for s in range(_B):
            a = jnp.maximum(row0, offs[s])
            b = jnp.minimum(row1, offs[s + 1])

            @pl.when(b > a)
            def _():
                def rbody(r, carry):
                    rr = r - row0
                    return tuple(
                        carry[j] + buf[rr, pl.ds(j * _L, _L)]
                        for j in range(_DV)
                    )

                run = lax.fori_loop(a, b, rbody, (zero,) * _DV)
                for j in range(_DV):
                    o = j * _L
                    acc[s, pl.ds(o, _L)] = acc[s, pl.ds(o, _L)] + run[j]

    def pair_body(i, c):
        k = i * 2
        for slot in range(2):
            @pl.when(k + slot < kw)
            def _():
                process(k + slot, slot)
        return c

    lax.fori_loop(0, (kw + 1) // 2, pair_body, 0)
    pltpu.sync_copy(acc, part_hbm.at[pl.ds(wid * _B, _B), :])


def _combine_body(part_hbm, len_hbm, out_hbm, idx_v, rows_v, len_v, out_v, sem):
    cid = lax.axis_index("c")
    sid = lax.axis_index("s")
    wid = sid * _NC + cid

    @pl.when(wid < _B)
    def _():
        lane = lax.iota(jnp.int32, _L)
        idx_v[pl.ds(0, _L)] = lane * _B + wid
        idx_v[pl.ds(_L, _L)] = (lane + _L) * _B + wid
        pltpu.async_copy(part_hbm.at[idx_v], rows_v, sem).wait()

        pltpu.sync_copy(len_hbm, len_v)
        lens = len_v[...]
        cnt = jnp.max(jnp.where(lane == wid, jnp.maximum(lens, 1), 0))
        cnt_vec = jnp.full((_L,), cnt, jnp.int32).astype(jnp.float32)
        recip = jnp.ones((_L,), jnp.float32) / cnt_vec

        def rbody(r, carry):
            return tuple(
                carry[j] + rows_v[r, pl.ds(j * _L, _L)] for j in range(_DV)
            )

        tot = lax.fori_loop(
            0, _NW, rbody, (jnp.zeros((_L,), jnp.float32),) * _DV
        )
        for j in range(_DV):
            out_v[0, pl.ds(j * _L, _L)] = tot[j] * recip
        pltpu.sync_copy(out_v, out_hbm.at[pl.ds(wid, 1), :])


_mesh = plsc.VectorSubcoreMesh(core_axis_name="c", subcore_axis_name="s")
_params = pltpu.CompilerParams(needs_layout_passes=False)

_sum_call = pl.kernel(
    _sum_body,
    out_type=jax.ShapeDtypeStruct((_NW * _B, _D), jnp.float32),
    mesh=_mesh,
    compiler_params=_params,
    scratch_types=[
        pltpu.VMEM((_L,), jnp.int32),
        pltpu.VMEM((_C, _D), jnp.float32),
        pltpu.VMEM((_C, _D), jnp.float32),
        pltpu.VMEM((_B, _D), jnp.float32),
        pltpu.SemaphoreType.DMA,
        pltpu.SemaphoreType.DMA,
    ],
)

_combine_call = pl.kernel(
    _combine_body,
    out_type=jax.ShapeDtypeStruct((_B, _D), jnp.float32),
    mesh=_mesh,
    compiler_params=_params,
    scratch_types=[
        pltpu.VMEM((_NW,), jnp.int32),
        pltpu.VMEM((_NW, _D), jnp.float32),
        pltpu.VMEM((_L,), jnp.int32),
        pltpu.VMEM((1, _D), jnp.float32),
        pltpu.SemaphoreType.DMA,
    ],
)


def kernel(x, batch_lengths):
    part = _sum_call(x, batch_lengths)
    return _combine_call(part, batch_lengths)
